# baseline (speedup 1.0000x reference)
import functools

import jax
import jax.numpy as jnp
from jax import lax
from jax.experimental import pallas as pl
from jax.experimental.pallas import tpu as pltpu
from jax.experimental.pallas import tpu_sc as plsc

N_NODES = 10000
N_EDGES = 160000
D = 256
DH = D // 2
NS = 16
CHUNK = 128
NCHUNK = 80
NBUF = 2
EPW = NCHUNK * CHUNK
E_PAD = NS * EPW
PAD_N = 10240
ROWS_PER_SUB = PAD_N // NS
ZROWS = 128

_sc_mesh = plsc.VectorSubcoreMesh(core_axis_name="c", subcore_axis_name="s")


@functools.partial(
    pl.kernel,
    out_type=jax.ShapeDtypeStruct((2, PAD_N, DH), jnp.float32),
    mesh=_sc_mesh,
    scratch_types=(
        [pltpu.VMEM((EPW,), jnp.int32)]
        + [pltpu.VMEM((CHUNK,), jnp.int32) for _ in range(NBUF)]
        + [pltpu.VMEM((CHUNK, DH), jnp.float32) for _ in range(NBUF)]
        + [pltpu.VMEM_SHARED((PAD_N, DH), jnp.float32)]
        + [pltpu.SemaphoreType.DMA for _ in range(2 * NBUF + 1)]
    ),
)
def _sc_aggregate(x2_hbm, src_hbm, dst_hbm, z_hbm,
                  o_hbm,
                  src_v, db0, db1, rb0, rb1,
                  agg_sh, sem0, sem1, dsem0, dsem1, isem):
    c = lax.axis_index("c")
    s = lax.axis_index("s")
    sems = (sem0, sem1)
    dsems = (dsem0, dsem1)
    dbs = (db0, db1)
    rbs = (rb0, rb1)
    x_ref = x2_hbm.at[c]
    out_ref = o_hbm.at[c]

    h_src = pltpu.async_copy(src_hbm.at[s], src_v, isem)

    @pl.loop(0, ROWS_PER_SUB // ZROWS)
    def _(k):
        pltpu.sync_copy(z_hbm,
                        agg_sh.at[pl.ds(s * ROWS_PER_SUB + k * ZROWS, ZROWS)])
    h_src.wait()
    plsc.subcore_barrier()

    @pl.loop(0, NCHUNK, step=NBUF)
    def _(j0):
        gh, dh = [], []
        for b in range(NBUF):
            dh.append(pltpu.async_copy(
                dst_hbm.at[s, pl.ds((j0 + b) * CHUNK, CHUNK)],
                dbs[b], dsems[b]))
            gh.append(pltpu.async_copy(
                x_ref.at[src_v.at[pl.ds((j0 + b) * CHUNK, CHUNK)]],
                rbs[b], sems[b]))
        for b in range(NBUF):
            gh[b].wait()
            dh[b].wait()
            pltpu.sync_copy(rbs[b], agg_sh.at[dbs[b]], add=True)

    plsc.subcore_barrier()
    pltpu.sync_copy(agg_sh.at[pl.ds(s * ROWS_PER_SUB, ROWS_PER_SUB)],
                    out_ref.at[pl.ds(s * ROWS_PER_SUB, ROWS_PER_SUB)])


BLK = 1000


def _mlp_body(x_ref, a0_ref, a1_ref, w1_ref, b1_ref, w2_ref, b2_ref, o_ref):
    h = x_ref[...]
    agg = jnp.concatenate([a0_ref[...], a1_ref[...]], axis=1)
    h = h + agg
    h1 = jnp.maximum(
        lax.dot_general(h, w1_ref[...], (((1,), (0,)), ((), ())),
                        preferred_element_type=jnp.float32,
                        precision=lax.Precision.HIGHEST) + b1_ref[...], 0.0)
    o_ref[...] = lax.dot_general(h1, w2_ref[...], (((1,), (0,)), ((), ())),
                                 preferred_element_type=jnp.float32,
                                 precision=lax.Precision.HIGHEST) + b2_ref[...]


def _mlp(xf, a0, a1, W1, b1, W2, b2):
    grid = (N_NODES // BLK,)
    return pl.pallas_call(
        _mlp_body,
        grid=grid,
        in_specs=[
            pl.BlockSpec((BLK, D), lambda i: (i, 0)),
            pl.BlockSpec((BLK, DH), lambda i: (i, 0)),
            pl.BlockSpec((BLK, DH), lambda i: (i, 0)),
            pl.BlockSpec((D, D), lambda i: (0, 0)),
            pl.BlockSpec((1, D), lambda i: (0, 0)),
            pl.BlockSpec((D, D), lambda i: (0, 0)),
            pl.BlockSpec((1, D), lambda i: (0, 0)),
        ],
        out_specs=pl.BlockSpec((BLK, D), lambda i: (i, 0)),
        out_shape=jax.ShapeDtypeStruct((N_NODES, D), jnp.float32),
    )(xf, a0, a1, W1, b1.reshape(1, D), W2, b2.reshape(1, D))


def kernel(x, edge_index, W1, b1, W2, b2):
    xf = x[0]
    ei = edge_index.astype(jnp.int32)
    src = jnp.concatenate(
        [ei[0], jnp.zeros((E_PAD - N_EDGES,), jnp.int32)]
    ).reshape(NS, EPW)
    dst = jnp.concatenate(
        [ei[1], jnp.full((E_PAD - N_EDGES,), N_NODES, jnp.int32)]
    ).reshape(NS, EPW)
    x2 = xf.reshape(N_NODES, 2, DH).transpose(1, 0, 2)
    z = jnp.zeros((ZROWS, DH), jnp.float32)
    agg = _sc_aggregate(x2, src, dst, z)
    a0 = agg[0, :N_NODES]
    a1 = agg[1, :N_NODES]
    out = _mlp(xf, a0, a1, W1, b1, W2, b2)
    return out.reshape(1, N_NODES, D)


# speedup vs baseline: 1.1906x; 1.1906x over previous
import functools

import jax
import jax.numpy as jnp
from jax import lax
from jax.experimental import pallas as pl
from jax.experimental.pallas import tpu as pltpu
from jax.experimental.pallas import tpu_sc as plsc

N_NODES = 10000
N_EDGES = 160000
D = 256
DH = D // 2
NS = 16
EPW = N_EDGES // NS
CHUNK = 80
NCHUNK = EPW // CHUNK
PAD_N = 10240
ROWS_PER_SUB = PAD_N // NS
ZROWS = 128

_sc_mesh = plsc.VectorSubcoreMesh(core_axis_name="c", subcore_axis_name="s")


@functools.partial(
    pl.kernel,
    out_type=(
        jax.ShapeDtypeStruct((PAD_N, DH), jnp.float32),
        jax.ShapeDtypeStruct((PAD_N, DH), jnp.float32),
    ),
    mesh=_sc_mesh,
    scratch_types=[
        pltpu.VMEM((CHUNK,), jnp.int32),
        pltpu.VMEM((CHUNK,), jnp.int32),
        pltpu.VMEM((CHUNK, DH), jnp.float32),
        pltpu.VMEM_SHARED((PAD_N, DH), jnp.float32),
        pltpu.SemaphoreType.DMA,
    ],
)
def _sc_aggregate(x0_hbm, x1_hbm, src_hbm, dst_hbm, z_hbm,
                  o0_hbm, o1_hbm,
                  src_v, dst_v, rows_v, agg_sh, sem):
    c = lax.axis_index("c")
    s = lax.axis_index("s")

    def run(x_ref, out_ref):
        @pl.loop(0, ROWS_PER_SUB // ZROWS)
        def _(k):
            pltpu.sync_copy(z_hbm,
                            agg_sh.at[pl.ds(s * ROWS_PER_SUB + k * ZROWS, ZROWS)])
        plsc.subcore_barrier()

        base = s * EPW

        @pl.loop(0, NCHUNK)
        def _(j):
            off = base + j * CHUNK
            pltpu.sync_copy(src_hbm.at[pl.ds(off, CHUNK)], src_v)
            pltpu.sync_copy(dst_hbm.at[pl.ds(off, CHUNK)], dst_v)
            pltpu.async_copy(x_ref.at[src_v], rows_v, sem).wait()
            pltpu.sync_copy(rows_v, agg_sh.at[dst_v], add=True)

        plsc.subcore_barrier()
        pltpu.sync_copy(agg_sh.at[pl.ds(s * ROWS_PER_SUB, ROWS_PER_SUB)],
                        out_ref.at[pl.ds(s * ROWS_PER_SUB, ROWS_PER_SUB)])

    @pl.when(c == 0)
    def _():
        run(x0_hbm, o0_hbm)

    @pl.when(c == 1)
    def _():
        run(x1_hbm, o1_hbm)


BLK = 1000


def _mlp_body(x_ref, a0_ref, a1_ref, w1_ref, b1_ref, w2_ref, b2_ref, o_ref):
    h = x_ref[...]
    agg = jnp.concatenate([a0_ref[...], a1_ref[...]], axis=1)
    h = h + agg
    h1 = jnp.maximum(
        lax.dot_general(h, w1_ref[...], (((1,), (0,)), ((), ())),
                        preferred_element_type=jnp.float32,
                        precision=lax.Precision.DEFAULT) + b1_ref[...], 0.0)
    o_ref[...] = lax.dot_general(h1, w2_ref[...], (((1,), (0,)), ((), ())),
                                 preferred_element_type=jnp.float32,
                                 precision=lax.Precision.DEFAULT) + b2_ref[...]


def _mlp(xf, a0, a1, W1, b1, W2, b2):
    grid = (N_NODES // BLK,)
    return pl.pallas_call(
        _mlp_body,
        grid=grid,
        in_specs=[
            pl.BlockSpec((BLK, D), lambda i: (i, 0)),
            pl.BlockSpec((BLK, DH), lambda i: (i, 0)),
            pl.BlockSpec((BLK, DH), lambda i: (i, 0)),
            pl.BlockSpec((D, D), lambda i: (0, 0)),
            pl.BlockSpec((1, D), lambda i: (0, 0)),
            pl.BlockSpec((D, D), lambda i: (0, 0)),
            pl.BlockSpec((1, D), lambda i: (0, 0)),
        ],
        out_specs=pl.BlockSpec((BLK, D), lambda i: (i, 0)),
        out_shape=jax.ShapeDtypeStruct((N_NODES, D), jnp.float32),
    )(xf, a0, a1, W1, b1.reshape(1, D), W2, b2.reshape(1, D))


def kernel(x, edge_index, W1, b1, W2, b2):
    xf = x[0]
    ei = edge_index.astype(jnp.int32)
    src = ei[0]
    dst = ei[1]
    x0 = xf[:, :DH]
    x1 = xf[:, DH:]
    z = jnp.zeros((ZROWS, DH), jnp.float32)
    a0, a1 = _sc_aggregate(x0, x1, src, dst, z)
    out = _mlp(xf, a0, a1, W1, b1, W2, b2)
    return out.reshape(1, N_NODES, D)


# speedup vs baseline: 1.8181x; 1.5270x over previous
import functools

import jax
import jax.numpy as jnp
from jax import lax
from jax.experimental import pallas as pl
from jax.experimental.pallas import tpu as pltpu
from jax.experimental.pallas import tpu_sc as plsc

N_NODES = 10000
N_EDGES = 160000
D = 256
DH = D // 2
NS = 16
EPW = N_EDGES // NS
CHUNK = 80
NCHUNK = EPW // CHUNK
PAD_N = 10240
ROWS_PER_SUB = PAD_N // NS
ZROWS = 128

_sc_mesh = plsc.VectorSubcoreMesh(core_axis_name="c", subcore_axis_name="s")


@functools.partial(
    pl.kernel,
    out_type=(
        jax.ShapeDtypeStruct((PAD_N, DH), jnp.float32),
        jax.ShapeDtypeStruct((PAD_N, DH), jnp.float32),
    ),
    mesh=_sc_mesh,
    scratch_types=[
        pltpu.VMEM((CHUNK,), jnp.int32),
        pltpu.VMEM((CHUNK,), jnp.int32),
        pltpu.VMEM((CHUNK,), jnp.int32),
        pltpu.VMEM((CHUNK,), jnp.int32),
        pltpu.VMEM((CHUNK, DH), jnp.float32),
        pltpu.VMEM((CHUNK, DH), jnp.float32),
        pltpu.VMEM_SHARED((PAD_N, DH), jnp.float32),
        pltpu.SemaphoreType.DMA,
        pltpu.SemaphoreType.DMA,
    ],
)
def _sc_aggregate(x0_hbm, x1_hbm, src_hbm, dst_hbm, z_hbm,
                  o0_hbm, o1_hbm,
                  src_v0, dst_v0, src_v1, dst_v1, rows_v0, rows_v1,
                  agg_sh, sem0, sem1):
    c = lax.axis_index("c")
    s = lax.axis_index("s")

    def run(x_ref, out_ref):
        @pl.loop(0, ROWS_PER_SUB // ZROWS)
        def _(k):
            pltpu.sync_copy(z_hbm,
                            agg_sh.at[pl.ds(s * ROWS_PER_SUB + k * ZROWS, ZROWS)])
        plsc.subcore_barrier()

        base = s * EPW

        def load_idx(j, sv, dv):
            off = base + j * CHUNK
            pltpu.sync_copy(src_hbm.at[pl.ds(off, CHUNK)], sv)
            pltpu.sync_copy(dst_hbm.at[pl.ds(off, CHUNK)], dv)

        load_idx(0, src_v0, dst_v0)
        g0 = pltpu.async_copy(x_ref.at[src_v0], rows_v0, sem0)

        @pl.loop(0, NCHUNK - 1, step=2)
        def _(j0):
            load_idx(j0 + 1, src_v1, dst_v1)
            pltpu.async_copy(x_ref.at[src_v1], rows_v1, sem1)
            pltpu.make_async_copy(x_ref.at[src_v0], rows_v0, sem0).wait()
            pltpu.sync_copy(rows_v0, agg_sh.at[dst_v0], add=True)
            load_idx(j0 + 2, src_v0, dst_v0)
            pltpu.async_copy(x_ref.at[src_v0], rows_v0, sem0)
            pltpu.make_async_copy(x_ref.at[src_v1], rows_v1, sem1).wait()
            pltpu.sync_copy(rows_v1, agg_sh.at[dst_v1], add=True)

        g0.wait()
        pltpu.sync_copy(rows_v0, agg_sh.at[dst_v0], add=True)

        plsc.subcore_barrier()
        pltpu.sync_copy(agg_sh.at[pl.ds(s * ROWS_PER_SUB, ROWS_PER_SUB)],
                        out_ref.at[pl.ds(s * ROWS_PER_SUB, ROWS_PER_SUB)])

    @pl.when(c == 0)
    def _():
        run(x0_hbm, o0_hbm)

    @pl.when(c == 1)
    def _():
        run(x1_hbm, o1_hbm)


BLK = 1000


def _mlp_body(x_ref, a0_ref, a1_ref, w1_ref, b1_ref, w2_ref, b2_ref, o_ref):
    h = x_ref[...]
    agg = jnp.concatenate([a0_ref[...], a1_ref[...]], axis=1)
    h = h + agg
    h1 = jnp.maximum(
        lax.dot_general(h, w1_ref[...], (((1,), (0,)), ((), ())),
                        preferred_element_type=jnp.float32,
                        precision=lax.Precision.DEFAULT) + b1_ref[...], 0.0)
    o_ref[...] = lax.dot_general(h1, w2_ref[...], (((1,), (0,)), ((), ())),
                                 preferred_element_type=jnp.float32,
                                 precision=lax.Precision.DEFAULT) + b2_ref[...]


def _mlp(xf, a0, a1, W1, b1, W2, b2):
    grid = (N_NODES // BLK,)
    return pl.pallas_call(
        _mlp_body,
        grid=grid,
        in_specs=[
            pl.BlockSpec((BLK, D), lambda i: (i, 0)),
            pl.BlockSpec((BLK, DH), lambda i: (i, 0)),
            pl.BlockSpec((BLK, DH), lambda i: (i, 0)),
            pl.BlockSpec((D, D), lambda i: (0, 0)),
            pl.BlockSpec((1, D), lambda i: (0, 0)),
            pl.BlockSpec((D, D), lambda i: (0, 0)),
            pl.BlockSpec((1, D), lambda i: (0, 0)),
        ],
        out_specs=pl.BlockSpec((BLK, D), lambda i: (i, 0)),
        out_shape=jax.ShapeDtypeStruct((N_NODES, D), jnp.float32),
    )(xf, a0, a1, W1, b1.reshape(1, D), W2, b2.reshape(1, D))


def kernel(x, edge_index, W1, b1, W2, b2):
    xf = x[0]
    ei = edge_index.astype(jnp.int32)
    src = ei[0]
    dst = ei[1]
    x0 = xf[:, :DH]
    x1 = xf[:, DH:]
    z = jnp.zeros((ZROWS, DH), jnp.float32)
    a0, a1 = _sc_aggregate(x0, x1, src, dst, z)
    out = _mlp(xf, a0, a1, W1, b1, W2, b2)
    return out.reshape(1, N_NODES, D)
